# baseline (speedup 1.0000x reference)
import jax
import jax.numpy as jnp
from jax import lax
from jax.experimental import pallas as pl
from jax.experimental.pallas import tpu as pltpu
from jax.experimental.pallas import tpu_sc as plsc

N = 10000
E = 320000
IN_DIM = 128
OUT_DIM = 32
NEG = 0.2
PAD = 48
NC = 2
NS = 16
NW = NC * NS
EPT = E // NW
CHUNK = 80
NCH = EPT // CHUNK
GPC = CHUNK // 16
NP = 10240
RPT = NP // NS
ZB = 128
RB = 2000
HGRID = N // RB
NBUF = 5
NSUP = NCH // NBUF


def _tc_prep_body(f_ref, w_ref, asrc_ref, adst_ref, g_ref, asd_ref):
    i = pl.program_id(0)
    xp = jnp.dot(f_ref[...], w_ref[...], preferred_element_type=jnp.float32)
    a_s = jnp.sum(xp * asrc_ref[...], axis=1)
    a_d = jnp.sum(xp * adst_ref[...], axis=1)
    scale = jnp.where(i < HGRID, 1.0, NEG)
    e = jnp.exp(scale * a_s)[:, None]
    zeros = jnp.zeros((RB, PAD - OUT_DIM - 1), jnp.float32)
    g_ref[...] = jnp.concatenate([xp * e, e, zeros], axis=1)
    asd_ref[...] = jnp.concatenate([a_s[:, None], a_d[:, None]], axis=1)


_tc_prep = pl.pallas_call(
    _tc_prep_body,
    grid=(2 * HGRID,),
    in_specs=[
        pl.BlockSpec((RB, IN_DIM), lambda i: (i % HGRID, 0)),
        pl.BlockSpec((IN_DIM, OUT_DIM), lambda i: (0, 0)),
        pl.BlockSpec((1, OUT_DIM), lambda i: (0, 0)),
        pl.BlockSpec((1, OUT_DIM), lambda i: (0, 0)),
    ],
    out_specs=[
        pl.BlockSpec((RB, PAD), lambda i: (i, 0)),
        pl.BlockSpec((RB, 2), lambda i: (i % HGRID, 0)),
    ],
    out_shape=[
        jax.ShapeDtypeStruct((2 * N, PAD), jnp.float32),
        jax.ShapeDtypeStruct((N, 2), jnp.float32),
    ],
)


def _sc_edge_body(a_s_hbm, a_d_hbm, pe_hbm, zeros_hbm, g_hbm,
                  out_hbm, a_s_v, a_d_v, pe_v, gidx_v, sidx_v,
                  rows_v, acc_sh, *sems):
    gsem = sems[:NBUF]
    ssem = sems[NBUF:]
    cid = lax.axis_index("c")
    sid = lax.axis_index("s")
    wid = cid * NS + sid

    for q in range(RPT // ZB):
        pltpu.async_copy(zeros_hbm,
                         acc_sh.at[pl.ds(sid * RPT + q * ZB, ZB)], gsem[q])
        pltpu.async_copy(zeros_hbm,
                         acc_sh.at[pl.ds(NP + sid * RPT + q * ZB, ZB)], ssem[q])
    pltpu.async_copy(a_s_hbm, a_s_v, ssem[NBUF - 1])
    pltpu.async_copy(a_d_hbm, a_d_v, gsem[NBUF - 1])
    pltpu.async_copy(pe_hbm.at[wid], pe_v, ssem[NBUF - 2])
    for q in range(RPT // ZB):
        pltpu.make_async_copy(
            zeros_hbm, acc_sh.at[pl.ds(sid * RPT + q * ZB, ZB)], gsem[q]).wait()
        pltpu.make_async_copy(
            zeros_hbm, acc_sh.at[pl.ds(NP + sid * RPT + q * ZB, ZB)], ssem[q]).wait()
    pltpu.make_async_copy(a_s_hbm, a_s_v, ssem[NBUF - 1]).wait()
    pltpu.make_async_copy(a_d_hbm, a_d_v, gsem[NBUF - 1]).wait()
    pltpu.make_async_copy(pe_hbm.at[wid], pe_v, ssem[NBUF - 2]).wait()

    def idx_body(ch, _):
        for gg in range(GPC):
            sl = pl.ds(gg * 16, 16)
            pe16 = pe_v[ch, sl]
            src16 = jnp.bitwise_and(pe16, 0xFFFF)
            dst16 = lax.shift_right_logical(pe16, 16)
            s = (plsc.load_gather(a_s_v, [src16])
                 + plsc.load_gather(a_d_v, [dst16]))
            neg = s <= 0
            gidx_v[ch, sl] = src16 + jnp.where(neg, N, 0)
            sidx_v[ch, sl] = dst16 + jnp.where(neg, NP, 0)
        return 0

    lax.fori_loop(0, NCH, idx_body, 0)
    plsc.subcore_barrier()

    def super_body(g, _):
        for b in range(NBUF):
            j = g * NBUF + b
            jprev = jnp.maximum(j - NBUF, 0)

            @pl.when(g > 0)
            def _wait_prev():
                pltpu.make_async_copy(
                    rows_v.at[b], acc_sh.at[sidx_v.at[jprev]], ssem[b]).wait()

            pltpu.async_copy(g_hbm.at[gidx_v.at[j]], rows_v.at[b], gsem[b])
        for b in range(NBUF):
            j = g * NBUF + b
            pltpu.make_async_copy(
                g_hbm.at[gidx_v.at[j]], rows_v.at[b], gsem[b]).wait()
            pltpu.async_copy(rows_v.at[b], acc_sh.at[sidx_v.at[j]], ssem[b],
                             add=True)
        return 0

    lax.fori_loop(0, NSUP, super_body, 0)
    for b in range(NBUF):
        j = (NSUP - 1) * NBUF + b
        pltpu.make_async_copy(
            rows_v.at[b], acc_sh.at[sidx_v.at[j]], ssem[b]).wait()
    plsc.subcore_barrier()
    pltpu.async_copy(acc_sh.at[pl.ds(sid * RPT, RPT)],
                     out_hbm.at[cid, pl.ds(sid * RPT, RPT)], gsem[0])
    pltpu.async_copy(acc_sh.at[pl.ds(NP + sid * RPT, RPT)],
                     out_hbm.at[cid, pl.ds(NP + sid * RPT, RPT)], gsem[1])
    pltpu.make_async_copy(acc_sh.at[pl.ds(sid * RPT, RPT)],
                          out_hbm.at[cid, pl.ds(sid * RPT, RPT)], gsem[0]).wait()
    pltpu.make_async_copy(acc_sh.at[pl.ds(NP + sid * RPT, RPT)],
                          out_hbm.at[cid, pl.ds(NP + sid * RPT, RPT)], gsem[1]).wait()


_sc_edge_cache = []


def _get_sc_edge():
    if not _sc_edge_cache:
        _sc_edge_cache.append(pl.kernel(
            _sc_edge_body,
            mesh=plsc.VectorSubcoreMesh(core_axis_name="c",
                                        subcore_axis_name="s"),
            compiler_params=pltpu.CompilerParams(needs_layout_passes=False,
                                                 use_tc_tiling_on_sc=False),
            out_type=jax.ShapeDtypeStruct((NC, 2 * NP, PAD), jnp.float32),
            scratch_types=[
                pltpu.VMEM((N,), jnp.float32),
                pltpu.VMEM((N,), jnp.float32),
                pltpu.VMEM((NCH, CHUNK), jnp.int32),
                pltpu.VMEM((NCH, CHUNK), jnp.int32),
                pltpu.VMEM((NCH, CHUNK), jnp.int32),
                pltpu.VMEM((NBUF, CHUNK, PAD), jnp.float32),
                pltpu.VMEM_SHARED((2 * NP, PAD), jnp.float32),
            ] + [pltpu.SemaphoreType.DMA] * (2 * NBUF),
        ))
    return _sc_edge_cache[0]


def _tc_finish_body(acc_ref, asd_ref, w_ref, h1_ref, h4_ref):
    s1 = acc_ref[0, 0] + acc_ref[1, 0]
    s2 = acc_ref[0, 1] + acc_ref[1, 1]
    a_d = asd_ref[:, 1:2]
    comb = jnp.exp(a_d) * s1 + jnp.exp(NEG * a_d) * s2
    num = comb[:, :OUT_DIM]
    den = comb[:, OUT_DIM:OUT_DIM + 1]
    h1 = num / (den + 1e-16)
    h1 = jnp.where(h1 > 0, h1, jnp.exp(h1) - 1.0)
    h1_ref[...] = h1
    h4_ref[...] = lax.dot_general(h1, w_ref[...], (((1,), (1,)), ((), ())),
                                  preferred_element_type=jnp.float32)


_tc_finish = pl.pallas_call(
    _tc_finish_body,
    grid=(HGRID,),
    in_specs=[
        pl.BlockSpec((2, 2, RB, PAD), lambda i: (0, 0, i, 0)),
        pl.BlockSpec((RB, 2), lambda i: (i, 0)),
        pl.BlockSpec((IN_DIM, OUT_DIM), lambda i: (0, 0)),
    ],
    out_specs=[
        pl.BlockSpec((RB, OUT_DIM), lambda i: (i, 0)),
        pl.BlockSpec((RB, IN_DIM), lambda i: (i, 0)),
    ],
    out_shape=[
        jax.ShapeDtypeStruct((N, OUT_DIM), jnp.float32),
        jax.ShapeDtypeStruct((N, IN_DIM), jnp.float32),
    ],
)


def kernel(features, edge_index, W1, att_src, att_dst):
    g_tab, asd = _tc_prep(features, W1, att_src[None, :], att_dst[None, :])
    packed = edge_index[0] + edge_index[1] * 65536
    pe3 = packed.reshape(NW, NCH, CHUNK)
    zeros = jnp.zeros((ZB, PAD), jnp.float32)
    a_s = asd[:, 0]
    a_d = asd[:, 1]
    acc = _get_sc_edge()(a_s, a_d, pe3, zeros, g_tab)
    acc4 = acc.reshape(NC, 2, NP, PAD)
    h1, h4 = _tc_finish(acc4, asd, W1)
    return (h1, h4)


# speedup vs baseline: 1.0187x; 1.0187x over previous
import jax
import jax.numpy as jnp
from jax import lax
from jax.experimental import pallas as pl
from jax.experimental.pallas import tpu as pltpu
from jax.experimental.pallas import tpu_sc as plsc

N = 10000
E = 320000
IN_DIM = 128
OUT_DIM = 32
NEG = 0.2
PAD = 48
NC = 2
NS = 16
NW = NC * NS
EPT = E // NW
CHUNK = 80
NCH = EPT // CHUNK
GPC = CHUNK // 16
NP = 10240
RPT = NP // NS
ZB = 128
RB = 2000
HGRID = N // RB
NBUF = 5
NSUP = NCH // NBUF


def _tc_prep_body(f_ref, w_ref, asrc_ref, adst_ref, g_ref, asd_ref):
    xp = jnp.dot(f_ref[...], w_ref[...], preferred_element_type=jnp.float32)
    a_s = jnp.sum(xp * asrc_ref[...], axis=1)
    a_d = jnp.sum(xp * adst_ref[...], axis=1)
    zeros = jnp.zeros((N, PAD - OUT_DIM - 1), jnp.float32)
    e1 = jnp.exp(a_s)[:, None]
    e2 = jnp.exp(NEG * a_s)[:, None]
    g_ref[0:N, :] = jnp.concatenate([xp * e1, e1, zeros], axis=1)
    g_ref[N:2 * N, :] = jnp.concatenate([xp * e2, e2, zeros], axis=1)
    asd_ref[...] = jnp.concatenate([a_s[:, None], a_d[:, None]], axis=1)


_tc_prep = pl.pallas_call(
    _tc_prep_body,
    grid=(1,),
    in_specs=[
        pl.BlockSpec((N, IN_DIM), lambda i: (0, 0)),
        pl.BlockSpec((IN_DIM, OUT_DIM), lambda i: (0, 0)),
        pl.BlockSpec((1, OUT_DIM), lambda i: (0, 0)),
        pl.BlockSpec((1, OUT_DIM), lambda i: (0, 0)),
    ],
    out_specs=[
        pl.BlockSpec((2 * N, PAD), lambda i: (0, 0)),
        pl.BlockSpec((N, 2), lambda i: (0, 0)),
    ],
    out_shape=[
        jax.ShapeDtypeStruct((2 * N, PAD), jnp.float32),
        jax.ShapeDtypeStruct((N, 2), jnp.float32),
    ],
)


def _sc_edge_body(a_s_hbm, a_d_hbm, pe_hbm, zeros_hbm, g_hbm,
                  out_hbm, a_s_v, a_d_v, pe_v, gidx_v, sidx_v,
                  rows_v, acc_sh, *sems):
    gsem = sems[:NBUF]
    ssem = sems[NBUF:]
    cid = lax.axis_index("c")
    sid = lax.axis_index("s")
    wid = cid * NS + sid

    for q in range(RPT // ZB):
        pltpu.async_copy(zeros_hbm,
                         acc_sh.at[pl.ds(sid * RPT + q * ZB, ZB)], gsem[q])
        pltpu.async_copy(zeros_hbm,
                         acc_sh.at[pl.ds(NP + sid * RPT + q * ZB, ZB)], ssem[q])
    pltpu.async_copy(a_s_hbm, a_s_v, ssem[NBUF - 1])
    pltpu.async_copy(a_d_hbm, a_d_v, gsem[NBUF - 1])
    pltpu.async_copy(pe_hbm.at[wid], pe_v, ssem[NBUF - 2])
    for q in range(RPT // ZB):
        pltpu.make_async_copy(
            zeros_hbm, acc_sh.at[pl.ds(sid * RPT + q * ZB, ZB)], gsem[q]).wait()
        pltpu.make_async_copy(
            zeros_hbm, acc_sh.at[pl.ds(NP + sid * RPT + q * ZB, ZB)], ssem[q]).wait()
    pltpu.make_async_copy(a_s_hbm, a_s_v, ssem[NBUF - 1]).wait()
    pltpu.make_async_copy(a_d_hbm, a_d_v, gsem[NBUF - 1]).wait()
    pltpu.make_async_copy(pe_hbm.at[wid], pe_v, ssem[NBUF - 2]).wait()

    def idx_body(ch, _):
        for gg in range(GPC):
            sl = pl.ds(gg * 16, 16)
            pe16 = pe_v[ch, sl]
            src16 = jnp.bitwise_and(pe16, 0xFFFF)
            dst16 = lax.shift_right_logical(pe16, 16)
            s = (plsc.load_gather(a_s_v, [src16])
                 + plsc.load_gather(a_d_v, [dst16]))
            neg = s <= 0
            gidx_v[ch, sl] = src16 + jnp.where(neg, N, 0)
            sidx_v[ch, sl] = dst16 + jnp.where(neg, NP, 0)
        return 0

    lax.fori_loop(0, NCH, idx_body, 0)
    plsc.subcore_barrier()

    def super_body(g, _):
        for b in range(NBUF):
            j = g * NBUF + b
            jprev = jnp.maximum(j - NBUF, 0)

            @pl.when(g > 0)
            def _wait_prev():
                pltpu.make_async_copy(
                    rows_v.at[b], acc_sh.at[sidx_v.at[jprev]], ssem[b]).wait()

            pltpu.async_copy(g_hbm.at[gidx_v.at[j]], rows_v.at[b], gsem[b])
        for b in range(NBUF):
            j = g * NBUF + b
            pltpu.make_async_copy(
                g_hbm.at[gidx_v.at[j]], rows_v.at[b], gsem[b]).wait()
            pltpu.async_copy(rows_v.at[b], acc_sh.at[sidx_v.at[j]], ssem[b],
                             add=True)
        return 0

    lax.fori_loop(0, NSUP, super_body, 0)
    for b in range(NBUF):
        j = (NSUP - 1) * NBUF + b
        pltpu.make_async_copy(
            rows_v.at[b], acc_sh.at[sidx_v.at[j]], ssem[b]).wait()
    plsc.subcore_barrier()
    pltpu.async_copy(acc_sh.at[pl.ds(sid * RPT, RPT)],
                     out_hbm.at[cid, pl.ds(sid * RPT, RPT)], gsem[0])
    pltpu.async_copy(acc_sh.at[pl.ds(NP + sid * RPT, RPT)],
                     out_hbm.at[cid, pl.ds(NP + sid * RPT, RPT)], gsem[1])
    pltpu.make_async_copy(acc_sh.at[pl.ds(sid * RPT, RPT)],
                          out_hbm.at[cid, pl.ds(sid * RPT, RPT)], gsem[0]).wait()
    pltpu.make_async_copy(acc_sh.at[pl.ds(NP + sid * RPT, RPT)],
                          out_hbm.at[cid, pl.ds(NP + sid * RPT, RPT)], gsem[1]).wait()


_sc_edge_cache = []


def _get_sc_edge():
    if not _sc_edge_cache:
        _sc_edge_cache.append(pl.kernel(
            _sc_edge_body,
            mesh=plsc.VectorSubcoreMesh(core_axis_name="c",
                                        subcore_axis_name="s"),
            compiler_params=pltpu.CompilerParams(needs_layout_passes=False,
                                                 use_tc_tiling_on_sc=False),
            out_type=jax.ShapeDtypeStruct((NC, 2 * NP, PAD), jnp.float32),
            scratch_types=[
                pltpu.VMEM((N,), jnp.float32),
                pltpu.VMEM((N,), jnp.float32),
                pltpu.VMEM((NCH, CHUNK), jnp.int32),
                pltpu.VMEM((NCH, CHUNK), jnp.int32),
                pltpu.VMEM((NCH, CHUNK), jnp.int32),
                pltpu.VMEM((NBUF, CHUNK, PAD), jnp.float32),
                pltpu.VMEM_SHARED((2 * NP, PAD), jnp.float32),
            ] + [pltpu.SemaphoreType.DMA] * (2 * NBUF),
        ))
    return _sc_edge_cache[0]


def _tc_finish_body(acc_ref, asd_ref, w_ref, h1_ref, h4_ref):
    s1 = acc_ref[0, 0, 0:N] + acc_ref[1, 0, 0:N]
    s2 = acc_ref[0, 1, 0:N] + acc_ref[1, 1, 0:N]
    a_d = asd_ref[:, 1:2]
    comb = jnp.exp(a_d) * s1 + jnp.exp(NEG * a_d) * s2
    num = comb[:, :OUT_DIM]
    den = comb[:, OUT_DIM:OUT_DIM + 1]
    h1 = num / (den + 1e-16)
    h1 = jnp.where(h1 > 0, h1, jnp.exp(h1) - 1.0)
    h1_ref[...] = h1
    h4_ref[...] = lax.dot_general(h1, w_ref[...], (((1,), (1,)), ((), ())),
                                  preferred_element_type=jnp.float32)


_tc_finish = pl.pallas_call(
    _tc_finish_body,
    grid=(1,),
    in_specs=[
        pl.BlockSpec((2, 2, NP, PAD), lambda i: (0, 0, 0, 0)),
        pl.BlockSpec((N, 2), lambda i: (0, 0)),
        pl.BlockSpec((IN_DIM, OUT_DIM), lambda i: (0, 0)),
    ],
    out_specs=[
        pl.BlockSpec((N, OUT_DIM), lambda i: (0, 0)),
        pl.BlockSpec((N, IN_DIM), lambda i: (0, 0)),
    ],
    out_shape=[
        jax.ShapeDtypeStruct((N, OUT_DIM), jnp.float32),
        jax.ShapeDtypeStruct((N, IN_DIM), jnp.float32),
    ],
)


def kernel(features, edge_index, W1, att_src, att_dst):
    g_tab, asd = _tc_prep(features, W1, att_src[None, :], att_dst[None, :])
    packed = edge_index[0] + edge_index[1] * 65536
    pe3 = packed.reshape(NW, NCH, CHUNK)
    zeros = jnp.zeros((ZB, PAD), jnp.float32)
    a_s = asd[:, 0]
    a_d = asd[:, 1]
    acc = _get_sc_edge()(a_s, a_d, pe3, zeros, g_tab)
    acc4 = acc.reshape(NC, 2, NP, PAD)
    h1, h4 = _tc_finish(acc4, asd, W1)
    return (h1, h4)


# speedup vs baseline: 1.1499x; 1.1287x over previous
import jax
import jax.numpy as jnp
from jax import lax
from jax.experimental import pallas as pl
from jax.experimental.pallas import tpu as pltpu
from jax.experimental.pallas import tpu_sc as plsc

N = 10000
E = 320000
IN_DIM = 128
OUT_DIM = 32
NEG = 0.2
DW = 16
NC = 2
NS = 16
NW = NC * NS
EPT = E // NW
CHUNK = 80
NCH = EPT // CHUNK
GPC = CHUNK // 16
NP = 10240
RPT = NP // NS
RB = 2000
NBUF = 5
NSUP = NCH // NBUF


def _tc_prep_body(f_ref, w_ref, asrc_ref, adst_ref, xp_ref, asd_ref):
    xp = jnp.dot(f_ref[...], w_ref[...], preferred_element_type=jnp.float32)
    xp_ref[...] = xp
    a_s = jnp.sum(xp * asrc_ref[...], axis=1)
    a_d = jnp.sum(xp * adst_ref[...], axis=1)
    asd_ref[...] = jnp.concatenate([a_s[:, None], a_d[:, None]], axis=1)


_tc_prep = pl.pallas_call(
    _tc_prep_body,
    grid=(N // RB,),
    in_specs=[
        pl.BlockSpec((RB, IN_DIM), lambda i: (i, 0)),
        pl.BlockSpec((IN_DIM, OUT_DIM), lambda i: (0, 0)),
        pl.BlockSpec((1, OUT_DIM), lambda i: (0, 0)),
        pl.BlockSpec((1, OUT_DIM), lambda i: (0, 0)),
    ],
    out_specs=[
        pl.BlockSpec((RB, OUT_DIM), lambda i: (i, 0)),
        pl.BlockSpec((RB, 2), lambda i: (i, 0)),
    ],
    out_shape=[
        jax.ShapeDtypeStruct((N, OUT_DIM), jnp.float32),
        jax.ShapeDtypeStruct((N, 2), jnp.float32),
    ],
)


def _sc_edge_body(a_s_hbm, a_d_hbm, src_hbm, dst_hbm, znum_hbm, zden_hbm,
                  zdr_hbm, xp_hbm, out_hbm, outden_hbm,
                  a_s_v, a_d_v, src_v, dst_v, w_v, rows_v, den_v,
                  acc_sh, accden_sh, *sems):
    gsem = sems[:NBUF]
    ssem = sems[NBUF:2 * NBUF]
    dsem = sems[2 * NBUF:]
    cid = lax.axis_index("c")
    sid = lax.axis_index("s")
    wid = cid * NS + sid

    pltpu.async_copy(znum_hbm, acc_sh.at[pl.ds(sid * RPT, RPT)], gsem[0])
    pltpu.async_copy(zden_hbm, accden_sh.at[pl.ds(sid * RPT, RPT)], gsem[1])
    pltpu.async_copy(zdr_hbm, den_v, gsem[2])

    pltpu.async_copy(a_s_hbm, a_s_v, ssem[0])
    pltpu.async_copy(a_d_hbm, a_d_v, ssem[1])
    pltpu.async_copy(src_hbm.at[wid], src_v, ssem[2])
    pltpu.async_copy(dst_hbm.at[wid], dst_v, ssem[3])
    pltpu.make_async_copy(znum_hbm, acc_sh.at[pl.ds(sid * RPT, RPT)], gsem[0]).wait()
    pltpu.make_async_copy(zden_hbm, accden_sh.at[pl.ds(sid * RPT, RPT)], gsem[1]).wait()
    pltpu.make_async_copy(zdr_hbm, den_v, gsem[2]).wait()
    pltpu.make_async_copy(a_s_hbm, a_s_v, ssem[0]).wait()
    pltpu.make_async_copy(a_d_hbm, a_d_v, ssem[1]).wait()
    pltpu.make_async_copy(src_hbm.at[wid], src_v, ssem[2]).wait()
    pltpu.make_async_copy(dst_hbm.at[wid], dst_v, ssem[3]).wait()
    plsc.subcore_barrier()

    lane = lax.iota(jnp.int32, 16)
    zero16 = jnp.zeros((16,), jnp.int32)

    def super_body(g, _):
        for b in range(NBUF):
            j = g * NBUF + b
            jprev = jnp.maximum(j - NBUF, 0)

            @pl.when(g > 0)
            def _wait_prev():
                pltpu.make_async_copy(
                    rows_v.at[b], acc_sh.at[dst_v.at[jprev]], ssem[b]).wait()
                pltpu.make_async_copy(
                    den_v.at[b], accden_sh.at[dst_v.at[jprev]], dsem[b]).wait()

            pltpu.async_copy(xp_hbm.at[src_v.at[j]], rows_v.at[b], gsem[b])

        for b in range(NBUF):
            j = g * NBUF + b
            for gg in range(GPC):
                src16 = src_v[j, pl.ds(gg * 16, 16)]
                dst16 = dst_v[j, pl.ds(gg * 16, 16)]
                s = (plsc.load_gather(a_s_v, [src16])
                     + plsc.load_gather(a_d_v, [dst16]))
                s = jnp.where(s > 0, s, NEG * s)
                w16 = jnp.exp(s)
                w_v[pl.ds(gg * 16, 16)] = w16
                plsc.store_scatter(den_v.at[b], [gg * 16 + lane, zero16], w16)
            pltpu.make_async_copy(
                xp_hbm.at[src_v.at[j]], rows_v.at[b], gsem[b]).wait()
            for gg in range(GPC):
                w16 = w_v[pl.ds(gg * 16, 16)]
                for k in range(16):
                    e = gg * 16 + k
                    wsp = w16[k]
                    for jj in range(OUT_DIM // 16):
                        sl = pl.ds(jj * 16, 16)
                        rows_v[b, e, sl] = rows_v[b, e, sl] * wsp
            pltpu.async_copy(rows_v.at[b], acc_sh.at[dst_v.at[j]], ssem[b],
                             add=True)
            pltpu.async_copy(den_v.at[b], accden_sh.at[dst_v.at[j]], dsem[b],
                             add=True)
        return 0

    lax.fori_loop(0, NSUP, super_body, 0)
    for b in range(NBUF):
        j = (NSUP - 1) * NBUF + b
        pltpu.make_async_copy(
            rows_v.at[b], acc_sh.at[dst_v.at[j]], ssem[b]).wait()
        pltpu.make_async_copy(
            den_v.at[b], accden_sh.at[dst_v.at[j]], dsem[b]).wait()
    plsc.subcore_barrier()
    pltpu.async_copy(acc_sh.at[pl.ds(sid * RPT, RPT)],
                     out_hbm.at[cid, pl.ds(sid * RPT, RPT)], gsem[0])
    pltpu.async_copy(accden_sh.at[pl.ds(sid * RPT, RPT)],
                     outden_hbm.at[cid, pl.ds(sid * RPT, RPT)], gsem[1])
    pltpu.make_async_copy(acc_sh.at[pl.ds(sid * RPT, RPT)],
                          out_hbm.at[cid, pl.ds(sid * RPT, RPT)], gsem[0]).wait()
    pltpu.make_async_copy(accden_sh.at[pl.ds(sid * RPT, RPT)],
                          outden_hbm.at[cid, pl.ds(sid * RPT, RPT)], gsem[1]).wait()


_sc_edge_cache = []


def _get_sc_edge():
    if not _sc_edge_cache:
        _sc_edge_cache.append(pl.kernel(
            _sc_edge_body,
            mesh=plsc.VectorSubcoreMesh(core_axis_name="c",
                                        subcore_axis_name="s"),
            compiler_params=pltpu.CompilerParams(needs_layout_passes=False,
                                                 use_tc_tiling_on_sc=False),
            out_type=[
                jax.ShapeDtypeStruct((NC, NP, OUT_DIM), jnp.float32),
                jax.ShapeDtypeStruct((NC, NP, DW), jnp.float32),
            ],
            scratch_types=[
                pltpu.VMEM((N,), jnp.float32),
                pltpu.VMEM((N,), jnp.float32),
                pltpu.VMEM((NCH, CHUNK), jnp.int32),
                pltpu.VMEM((NCH, CHUNK), jnp.int32),
                pltpu.VMEM((CHUNK,), jnp.float32),
                pltpu.VMEM((NBUF, CHUNK, OUT_DIM), jnp.float32),
                pltpu.VMEM((NBUF, CHUNK, DW), jnp.float32),
                pltpu.VMEM_SHARED((NP, OUT_DIM), jnp.float32),
                pltpu.VMEM_SHARED((NP, DW), jnp.float32),
            ] + [pltpu.SemaphoreType.DMA] * (3 * NBUF),
        ))
    return _sc_edge_cache[0]


def _tc_finish_body(acc_ref, accden_ref, w_ref, h1_ref, h4_ref):
    num = acc_ref[0] + acc_ref[1]
    den = accden_ref[0] + accden_ref[1]
    h1 = num / (den[:, 0:1] + 1e-16)
    h1 = jnp.where(h1 > 0, h1, jnp.exp(h1) - 1.0)
    h1_ref[...] = h1
    h4_ref[...] = lax.dot_general(h1, w_ref[...], (((1,), (1,)), ((), ())),
                                  preferred_element_type=jnp.float32)


_tc_finish = pl.pallas_call(
    _tc_finish_body,
    grid=(N // RB,),
    in_specs=[
        pl.BlockSpec((2, RB, OUT_DIM), lambda i: (0, i, 0)),
        pl.BlockSpec((2, RB, DW), lambda i: (0, i, 0)),
        pl.BlockSpec((IN_DIM, OUT_DIM), lambda i: (0, 0)),
    ],
    out_specs=[
        pl.BlockSpec((RB, OUT_DIM), lambda i: (i, 0)),
        pl.BlockSpec((RB, IN_DIM), lambda i: (i, 0)),
    ],
    out_shape=[
        jax.ShapeDtypeStruct((N, OUT_DIM), jnp.float32),
        jax.ShapeDtypeStruct((N, IN_DIM), jnp.float32),
    ],
)


def kernel(features, edge_index, W1, att_src, att_dst):
    xp, asd = _tc_prep(features, W1, att_src[None, :], att_dst[None, :])
    src3 = edge_index[0].reshape(NW, NCH, CHUNK)
    dst3 = edge_index[1].reshape(NW, NCH, CHUNK)
    znum = jnp.zeros((RPT, OUT_DIM), jnp.float32)
    zden = jnp.zeros((RPT, DW), jnp.float32)
    zdr = jnp.zeros((NBUF, CHUNK, DW), jnp.float32)
    a_s = asd[:, 0]
    a_d = asd[:, 1]
    acc, accden = _get_sc_edge()(a_s, a_d, src3, dst3, znum, zden, zdr, xp)
    h1, h4 = _tc_finish(acc, accden, W1)
    return (h1, h4)
